# baseline (speedup 1.0000x reference)
import functools

import jax
import jax.numpy as jnp
from jax.experimental import pallas as pl
from jax.experimental.pallas import tpu as pltpu

_TEMPERATURE = 0.5
_EPS = 1e-10


def _tc_body(l_ref, u_ref, hard_ref, y_ref):
    l = l_ref[...]
    u = u_ref[...]
    g = -jnp.log(-jnp.log(u + _EPS) + _EPS)
    s = (l + g) * (1.0 / _TEMPERATURE)
    lane = jax.lax.broadcasted_iota(jnp.int32, s.shape, 1)
    even = (lane % 2) == 0
    d_f = jnp.roll(s, -1, axis=1) - s
    d = jnp.where(even, d_f, -jnp.roll(d_f, 1, axis=1))
    y = 1.0 / (1.0 + jnp.exp(d))
    f_le = jnp.where(d <= 0.0, 1.0, 0.0)
    f_lt = jnp.where(d < 0.0, 1.0, 0.0)
    hard_ref[...] = jnp.where(even, f_le, f_lt)
    y_ref[...] = y


def _tc_gumbel(lv, uv, block_rows=256, interpret=False):
    m, n2 = lv.shape
    grid = (m // block_rows,)
    spec = pl.BlockSpec((block_rows, n2), lambda i: (i, 0))
    out = pl.pallas_call(
        _tc_body,
        grid=grid,
        in_specs=[spec, spec],
        out_specs=[spec, spec],
        out_shape=[
            jax.ShapeDtypeStruct((m, n2), jnp.float32),
            jax.ShapeDtypeStruct((m, n2), jnp.float32),
        ],
        interpret=interpret,
    )(lv, uv)
    return out


def kernel(logits, u):
    m, n, two = logits.shape
    lv = logits.reshape(m, n * two)
    uv = u.reshape(m, n * two)
    hard, y = _tc_gumbel(lv, uv)
    return (hard.reshape(m, n, two), y.reshape(m, n, two))


# speedup vs baseline: 6.9443x; 6.9443x over previous
import functools

import jax
import jax.numpy as jnp
from jax.experimental import pallas as pl
from jax.experimental.pallas import tpu as pltpu

_TEMPERATURE = 0.5
_EPS = 1e-10


def _tc_body(l_ref, u_ref, hard_ref, y_ref):
    l = l_ref[...]
    u = u_ref[...]
    g = -jnp.log(-jnp.log(u + _EPS) + _EPS)
    s = (l + g) * (1.0 / _TEMPERATURE)
    row = jax.lax.broadcasted_iota(jnp.int32, s.shape, 0)
    even = (row % 2) == 0
    d_f = jnp.roll(s, -1, axis=0) - s
    d = jnp.where(even, d_f, -jnp.roll(d_f, 1, axis=0))
    y = 1.0 / (1.0 + jnp.exp(d))
    f_le = jnp.where(d <= 0.0, 1.0, 0.0)
    f_lt = jnp.where(d < 0.0, 1.0, 0.0)
    hard_ref[...] = jnp.where(even, f_le, f_lt)
    y_ref[...] = y


def _tc_gumbel(lv, uv, block_rows=2048, interpret=False):
    m, n = lv.shape
    grid = (m // block_rows,)
    spec = pl.BlockSpec((block_rows, n), lambda i: (i, 0))
    out = pl.pallas_call(
        _tc_body,
        grid=grid,
        in_specs=[spec, spec],
        out_specs=[spec, spec],
        out_shape=[
            jax.ShapeDtypeStruct((m, n), jnp.float32),
            jax.ShapeDtypeStruct((m, n), jnp.float32),
        ],
        interpret=interpret,
    )(lv, uv)
    return out


def _to_rows(x):
    m, n, two = x.shape
    nb = n // 128
    return x.reshape(m, nb, 128, two).transpose(0, 1, 3, 2).reshape(m * nb * two, 128)


def _from_rows(x, m, n, two):
    nb = n // 128
    return x.reshape(m, nb, two, 128).transpose(0, 1, 3, 2).reshape(m, n, two)


def kernel(logits, u):
    m, n, two = logits.shape
    lv = _to_rows(logits)
    uv = _to_rows(u)
    hard, y = _tc_gumbel(lv, uv)
    return (_from_rows(hard, m, n, two), _from_rows(y, m, n, two))


# speedup vs baseline: 8.1726x; 1.1769x over previous
import functools

import jax
import jax.numpy as jnp
from jax.experimental import pallas as pl
from jax.experimental.pallas import tpu as pltpu

_TEMPERATURE = 0.5
_EPS = 1e-10


def _tc_body(l_ref, u_ref, hard_ref, y_ref):
    l = l_ref[...]
    u = u_ref[...]
    g = -jnp.log(-jnp.log(u + _EPS) + _EPS)
    s = (l + g) * (1.0 / _TEMPERATURE)
    row = jax.lax.broadcasted_iota(jnp.int32, s.shape, 0)
    even = (row % 2) == 0
    d_f = jnp.roll(s, -1, axis=0) - s
    d = jnp.where(even, d_f, -jnp.roll(d_f, 1, axis=0))
    y = 1.0 / (1.0 + jnp.exp(d))
    f_le = jnp.where(d <= 0.0, 1.0, 0.0)
    f_lt = jnp.where(d < 0.0, 1.0, 0.0)
    hard_ref[...] = jnp.where(even, f_le, f_lt)
    y_ref[...] = y


def _tc_gumbel(lv, uv, block_rows=4096, interpret=False):
    m, n = lv.shape
    grid = (m // block_rows,)
    spec = pl.BlockSpec((block_rows, n), lambda i: (i, 0))
    out = pl.pallas_call(
        _tc_body,
        grid=grid,
        in_specs=[spec, spec],
        out_specs=[spec, spec],
        out_shape=[
            jax.ShapeDtypeStruct((m, n), jnp.float32),
            jax.ShapeDtypeStruct((m, n), jnp.float32),
        ],
        interpret=interpret,
    )(lv, uv)
    return out


def _to_rows(x):
    m, n, two = x.shape
    nb = n // 128
    return x.reshape(m, nb, 128, two).transpose(0, 1, 3, 2).reshape(m * nb * two, 128)


def _from_rows(x, m, n, two):
    nb = n // 128
    return x.reshape(m, nb, two, 128).transpose(0, 1, 3, 2).reshape(m, n, two)


def kernel(logits, u):
    m, n, two = logits.shape
    lv = _to_rows(logits)
    uv = _to_rows(u)
    hard, y = _tc_gumbel(lv, uv)
    return (_from_rows(hard, m, n, two), _from_rows(y, m, n, two))


# speedup vs baseline: 8.8975x; 1.0887x over previous
import functools

import jax
import jax.numpy as jnp
from jax.experimental import pallas as pl
from jax.experimental.pallas import tpu as pltpu

_TEMPERATURE = 0.5
_EPS = 1e-10


def _tc_body(l_ref, u_ref, hard_ref, y_ref):
    l = l_ref[...]
    u = u_ref[...]
    g = -jnp.log(-jnp.log(u + _EPS) + _EPS)
    s = (l + g) * (1.0 / _TEMPERATURE)
    row = jax.lax.broadcasted_iota(jnp.int32, s.shape, 0)
    even = (row % 2) == 0
    d_f = jnp.roll(s, -1, axis=0) - s
    d = jnp.where(even, d_f, -jnp.roll(d_f, 1, axis=0))
    y = 1.0 / (1.0 + jnp.exp(d))
    f_le = jnp.where(d <= 0.0, 1.0, 0.0)
    f_lt = jnp.where(d < 0.0, 1.0, 0.0)
    hard_ref[...] = jnp.where(even, f_le, f_lt)
    y_ref[...] = y


def _tc_gumbel(lv, uv, block_rows=8192, interpret=False):
    m, n = lv.shape
    grid = (m // block_rows,)
    spec = pl.BlockSpec((block_rows, n), lambda i: (i, 0))
    out = pl.pallas_call(
        _tc_body,
        grid=grid,
        in_specs=[spec, spec],
        out_specs=[spec, spec],
        out_shape=[
            jax.ShapeDtypeStruct((m, n), jnp.float32),
            jax.ShapeDtypeStruct((m, n), jnp.float32),
        ],
        interpret=interpret,
    )(lv, uv)
    return out


def _to_rows(x):
    m, n, two = x.shape
    nb = n // 128
    return x.reshape(m, nb, 128, two).transpose(0, 1, 3, 2).reshape(m * nb * two, 128)


def _from_rows(x, m, n, two):
    nb = n // 128
    return x.reshape(m, nb, two, 128).transpose(0, 1, 3, 2).reshape(m, n, two)


def kernel(logits, u):
    m, n, two = logits.shape
    lv = _to_rows(logits)
    uv = _to_rows(u)
    hard, y = _tc_gumbel(lv, uv)
    return (_from_rows(hard, m, n, two), _from_rows(y, m, n, two))
